# baseline (speedup 1.0000x reference)
import functools

import jax
import jax.numpy as jnp
from jax import lax
from jax.experimental import pallas as pl
from jax.experimental.pallas import tpu as pltpu
from jax.experimental.pallas import tpu_sc as plsc

NUM_ITEMS = 1000000
EMBED_DIM = 64
H1 = 128
H2 = 64
BATCH = 16384
HIST = 50
HIST_PAD = 56
TOTAL_PAD = BATCH * HIST_PAD

NC = 2
NS = 16
NW = NC * NS
B_PER_W = TOTAL_PAD // NW
CHUNK = 896
N_CHUNKS = B_PER_W // CHUNK

NBLK = 8192


def _mlp_t_body(xT_ref, w1T_ref, b1_ref, w2_ref, b2_ref, o_ref):
    xT = xT_ref[...].astype(jnp.bfloat16)
    w1T = w1T_ref[...].astype(jnp.bfloat16)
    hT = lax.dot_general(
        w1T, xT, (((1,), (0,)), ((), ())),
        preferred_element_type=jnp.float32,
    )
    hT = jnp.maximum(hT + b1_ref[...], 0.0).astype(jnp.bfloat16)
    w2 = w2_ref[...].astype(jnp.bfloat16)
    out = lax.dot_general(
        hT, w2, (((0,), (0,)), ((), ())),
        preferred_element_type=jnp.float32,
    )
    out = jnp.maximum(out + b2_ref[...], 0.0)
    o_ref[:, :H2] = out


def _tc_mlp_table(embT, W1, b1, W2, b2):
    grid = (pl.cdiv(NUM_ITEMS, NBLK),)
    return pl.pallas_call(
        _mlp_t_body,
        grid=grid,
        in_specs=[
            pl.BlockSpec((EMBED_DIM, NBLK), lambda i: (0, i)),
            pl.BlockSpec((H1, EMBED_DIM), lambda i: (0, 0)),
            pl.BlockSpec((H1, 1), lambda i: (0, 0)),
            pl.BlockSpec((H1, H2), lambda i: (0, 0)),
            pl.BlockSpec((1, H2), lambda i: (0, 0)),
        ],
        out_specs=pl.BlockSpec((NBLK, 2 * H2), lambda i: (i, 0)),
        out_shape=jax.ShapeDtypeStruct(
            ((NUM_ITEMS + NBLK - 1) // NBLK * NBLK, 2 * H2),
            jnp.float32),
        compiler_params=pltpu.CompilerParams(
            dimension_semantics=("arbitrary",),
        ),
    )(embT, W1.T, b1.reshape(H1, 1), W2, b2.reshape(1, H2))


def _sc_gather(table, idx):
    mesh = plsc.VectorSubcoreMesh(core_axis_name="c", subcore_axis_name="s")

    @functools.partial(
        pl.kernel,
        out_type=jax.ShapeDtypeStruct((TOTAL_PAD, 2 * H2), jnp.float32),
        mesh=mesh,
        scratch_types=[
            pltpu.VMEM((CHUNK,), jnp.int32),
            pltpu.VMEM((CHUNK, H2), jnp.float32),
            pltpu.SemaphoreType.DMA,
        ],
        compiler_params=pltpu.CompilerParams(use_tc_tiling_on_sc=False),
    )
    def gather_kernel(table_hbm, idx_hbm, out_hbm, idx_v, rows_v, sem):
        wid = lax.axis_index("s") * NC + lax.axis_index("c")
        base = wid * B_PER_W

        def body(g, carry):
            off = base + g * CHUNK
            pltpu.sync_copy(idx_hbm.at[pl.ds(off, CHUNK)], idx_v)
            pltpu.async_copy(table_hbm.at[idx_v], rows_v, sem).wait()
            pltpu.sync_copy(
                rows_v, out_hbm.at[pl.ds(off, CHUNK), pl.ds(0, H2)])
            return carry

        lax.fori_loop(0, N_CHUNKS, body, 0)

    return gather_kernel(table, idx)


def kernel(batch, emb, W1, b1, W2, b2):
    idx = jnp.pad(batch.astype(jnp.int32) * 2,
                  ((0, 0), (0, HIST_PAD - HIST))).reshape(-1)
    table_out = _tc_mlp_table(emb.T, W1, b1, W2, b2)
    table_rows = table_out.reshape(-1, H2)
    out_pad = _sc_gather(table_rows, idx)
    return out_pad.reshape(BATCH, HIST_PAD, 2 * H2)[:, :HIST, :H2]


# speedup vs baseline: 4.1540x; 4.1540x over previous
import functools

import jax
import jax.numpy as jnp
from jax import lax
from jax.experimental import pallas as pl
from jax.experimental.pallas import tpu as pltpu
from jax.experimental.pallas import tpu_sc as plsc

NUM_ITEMS = 1000000
EMBED_DIM = 64
H1 = 128
H2 = 64
BATCH = 16384
HIST = 50
HIST_PAD = 56
TOTAL_PAD = BATCH * HIST_PAD

NC = 2
NS = 16
NW = NC * NS
B_PER_W = TOTAL_PAD // NW
CHUNK = 896
N_CHUNKS = B_PER_W // CHUNK

NBLK = 8192


def _mlp_t_body(xT_ref, w1T_ref, b1_ref, w2_ref, b2_ref, o_ref):
    xT = xT_ref[...].astype(jnp.bfloat16)
    w1T = w1T_ref[...].astype(jnp.bfloat16)
    hT = lax.dot_general(
        w1T, xT, (((1,), (0,)), ((), ())),
        preferred_element_type=jnp.float32,
    )
    hT = jnp.maximum(hT + b1_ref[...], 0.0).astype(jnp.bfloat16)
    w2 = w2_ref[...].astype(jnp.bfloat16)
    out = lax.dot_general(
        hT, w2, (((0,), (0,)), ((), ())),
        preferred_element_type=jnp.float32,
    )
    out = jnp.maximum(out + b2_ref[...], 0.0)
    o_ref[:, :H2] = out


def _tc_mlp_table(embT, W1, b1, W2, b2):
    grid = (pl.cdiv(NUM_ITEMS, NBLK),)
    return pl.pallas_call(
        _mlp_t_body,
        grid=grid,
        in_specs=[
            pl.BlockSpec((EMBED_DIM, NBLK), lambda i: (0, i)),
            pl.BlockSpec((H1, EMBED_DIM), lambda i: (0, 0)),
            pl.BlockSpec((H1, 1), lambda i: (0, 0)),
            pl.BlockSpec((H1, H2), lambda i: (0, 0)),
            pl.BlockSpec((1, H2), lambda i: (0, 0)),
        ],
        out_specs=pl.BlockSpec((NBLK, 2 * H2), lambda i: (i, 0)),
        out_shape=jax.ShapeDtypeStruct(
            ((NUM_ITEMS + NBLK - 1) // NBLK * NBLK, 2 * H2),
            jnp.float32),
        compiler_params=pltpu.CompilerParams(
            dimension_semantics=("arbitrary",),
        ),
    )(embT, W1.T, b1.reshape(H1, 1), W2, b2.reshape(1, H2))


def _sc_gather(table, idx):
    mesh = plsc.VectorSubcoreMesh(core_axis_name="c", subcore_axis_name="s")

    @functools.partial(
        pl.kernel,
        out_type=jax.ShapeDtypeStruct((TOTAL_PAD, 2 * H2), jnp.float32),
        mesh=mesh,
        scratch_types=[
            pltpu.VMEM((CHUNK,), jnp.int32),
            pltpu.VMEM((CHUNK, H2), jnp.float32),
            pltpu.SemaphoreType.DMA,
        ],
        compiler_params=pltpu.CompilerParams(use_tc_tiling_on_sc=False),
    )
    def gather_kernel(table_hbm, idx_hbm, out_hbm, idx_v, rows_v, sem):
        wid = lax.axis_index("s") * NC + lax.axis_index("c")
        base = wid * B_PER_W

        def body(g, carry):
            off = base + g * CHUNK
            pltpu.sync_copy(idx_hbm.at[pl.ds(off, CHUNK)], idx_v)
            pltpu.async_copy(table_hbm.at[idx_v], rows_v, sem).wait()
            pltpu.sync_copy(
                rows_v, out_hbm.at[pl.ds(off, CHUNK), pl.ds(0, H2)])
            return carry

        lax.fori_loop(0, N_CHUNKS, body, 0)

    return gather_kernel(table, idx)


def kernel(batch, emb, W1, b1, W2, b2):
    idx = jnp.pad(batch.astype(jnp.int32) * 2,
                  ((0, 0), (0, HIST_PAD - HIST)), mode="edge").reshape(-1)
    table_out = _tc_mlp_table(emb.T, W1, b1, W2, b2)
    table_rows = table_out.reshape(-1, H2)
    out_pad = _sc_gather(table_rows, idx)
    return out_pad.reshape(BATCH, HIST_PAD, 2 * H2)[:, :HIST, :H2]


# speedup vs baseline: 4.3893x; 1.0567x over previous
import functools

import jax
import jax.numpy as jnp
from jax import lax
from jax.experimental import pallas as pl
from jax.experimental.pallas import tpu as pltpu
from jax.experimental.pallas import tpu_sc as plsc

NUM_ITEMS = 1000000
EMBED_DIM = 64
H1 = 128
H2 = 64
BATCH = 16384
HIST = 50
HIST_PAD = 56
TOTAL_PAD = BATCH * HIST_PAD

NC = 2
NS = 16
NW = NC * NS
B_PER_W = TOTAL_PAD // NW
CHUNK = 896
N_CHUNKS = B_PER_W // CHUNK

NBLK = 4096
N_TBLK = 123
HALF = N_TBLK * NBLK


def _mlp_t_body(xTa_ref, xTb_ref, w1T_ref, b1_ref, w2_ref, b2_ref, o_ref):
    xT = jnp.concatenate(
        [xTa_ref[...], xTb_ref[...]], axis=1).astype(jnp.bfloat16)
    w1T = w1T_ref[...].astype(jnp.bfloat16)
    hT = lax.dot_general(
        w1T, xT, (((1,), (0,)), ((), ())),
        preferred_element_type=jnp.float32,
    )
    hT = jnp.maximum(hT + b1_ref[...], 0.0).astype(jnp.bfloat16)
    w2 = w2_ref[...].astype(jnp.bfloat16)
    out = lax.dot_general(
        hT, w2, (((0,), (0,)), ((), ())),
        preferred_element_type=jnp.float32,
    )
    out = jnp.maximum(out + b2_ref[...], 0.0)
    o_ref[...] = jnp.concatenate([out[:NBLK], out[NBLK:]], axis=1)


def _tc_mlp_table(embT, W1, b1, W2, b2):
    grid = (N_TBLK,)
    return pl.pallas_call(
        _mlp_t_body,
        grid=grid,
        in_specs=[
            pl.BlockSpec((EMBED_DIM, NBLK), lambda i: (0, i)),
            pl.BlockSpec((EMBED_DIM, NBLK),
                         lambda i: (0, jnp.minimum(i + N_TBLK,
                                                   2 * N_TBLK - 2))),
            pl.BlockSpec((H1, EMBED_DIM), lambda i: (0, 0)),
            pl.BlockSpec((H1, 1), lambda i: (0, 0)),
            pl.BlockSpec((H1, H2), lambda i: (0, 0)),
            pl.BlockSpec((1, H2), lambda i: (0, 0)),
        ],
        out_specs=pl.BlockSpec((NBLK, 2 * H2), lambda i: (i, 0)),
        out_shape=jax.ShapeDtypeStruct((HALF, 2 * H2), jnp.float32),
        compiler_params=pltpu.CompilerParams(
            dimension_semantics=("arbitrary",),
        ),
    )(embT, embT, W1.T, b1.reshape(H1, 1), W2, b2.reshape(1, H2))


def _sc_gather(table, idx):
    mesh = plsc.VectorSubcoreMesh(core_axis_name="c", subcore_axis_name="s")

    @functools.partial(
        pl.kernel,
        out_type=jax.ShapeDtypeStruct((TOTAL_PAD, 2 * H2), jnp.float32),
        mesh=mesh,
        scratch_types=[
            pltpu.VMEM((CHUNK,), jnp.int32),
            pltpu.VMEM((CHUNK, H2), jnp.float32),
            pltpu.SemaphoreType.DMA,
        ],
        compiler_params=pltpu.CompilerParams(use_tc_tiling_on_sc=False),
    )
    def gather_kernel(table_hbm, idx_hbm, out_hbm, idx_v, rows_v, sem):
        wid = lax.axis_index("s") * NC + lax.axis_index("c")
        base = wid * B_PER_W

        def body(g, carry):
            off = base + g * CHUNK
            pltpu.sync_copy(idx_hbm.at[pl.ds(off, CHUNK)], idx_v)
            pltpu.async_copy(table_hbm.at[idx_v], rows_v, sem).wait()
            pltpu.sync_copy(
                rows_v, out_hbm.at[pl.ds(off, CHUNK), pl.ds(0, H2)])
            return carry

        lax.fori_loop(0, N_CHUNKS, body, 0)

    return gather_kernel(table, idx)


def kernel(batch, emb, W1, b1, W2, b2):
    r = batch.astype(jnp.int32)
    remapped = jnp.where(r < HALF, r * 2, (r - HALF) * 2 + 1)
    idx = jnp.pad(remapped, ((0, 0), (0, HIST_PAD - HIST)),
                  mode="edge").reshape(-1)
    table_out = _tc_mlp_table(emb.T, W1, b1, W2, b2)
    table_rows = table_out.reshape(-1, H2)
    out_pad = _sc_gather(table_rows, idx)
    return out_pad.reshape(BATCH, HIST_PAD, 2 * H2)[:, :HIST, :H2]


# speedup vs baseline: 4.5689x; 1.0409x over previous
import functools

import jax
import jax.numpy as jnp
from jax import lax
from jax.experimental import pallas as pl
from jax.experimental.pallas import tpu as pltpu
from jax.experimental.pallas import tpu_sc as plsc

NUM_ITEMS = 1000000
EMBED_DIM = 64
H1 = 128
H2 = 64
BATCH = 16384
HIST = 50
HIST_PAD = 56
TOTAL_PAD = BATCH * HIST_PAD

NC = 2
NS = 16
NW = NC * NS
B_PER_W = TOTAL_PAD // NW
CHUNK = 1792
N_CHUNKS = B_PER_W // CHUNK

NBLK = 4096
N_TBLK = 123
HALF = N_TBLK * NBLK


def _mlp_t_body(xTa_ref, xTb_ref, w1T_ref, b1_ref, w2_ref, b2_ref, o_ref):
    xT = jnp.concatenate(
        [xTa_ref[...], xTb_ref[...]], axis=1).astype(jnp.bfloat16)
    w1T = w1T_ref[...].astype(jnp.bfloat16)
    hT = lax.dot_general(
        w1T, xT, (((1,), (0,)), ((), ())),
        preferred_element_type=jnp.float32,
    )
    hT = jnp.maximum(hT + b1_ref[...], 0.0).astype(jnp.bfloat16)
    w2 = w2_ref[...].astype(jnp.bfloat16)
    out = lax.dot_general(
        hT, w2, (((0,), (0,)), ((), ())),
        preferred_element_type=jnp.float32,
    )
    out = jnp.maximum(out + b2_ref[...], 0.0)
    o_ref[...] = jnp.concatenate([out[:NBLK], out[NBLK:]], axis=1)


def _tc_mlp_table(embT, W1, b1, W2, b2):
    grid = (N_TBLK,)
    return pl.pallas_call(
        _mlp_t_body,
        grid=grid,
        in_specs=[
            pl.BlockSpec((EMBED_DIM, NBLK), lambda i: (0, i)),
            pl.BlockSpec((EMBED_DIM, NBLK),
                         lambda i: (0, jnp.minimum(i + N_TBLK,
                                                   2 * N_TBLK - 2))),
            pl.BlockSpec((H1, EMBED_DIM), lambda i: (0, 0)),
            pl.BlockSpec((H1, 1), lambda i: (0, 0)),
            pl.BlockSpec((H1, H2), lambda i: (0, 0)),
            pl.BlockSpec((1, H2), lambda i: (0, 0)),
        ],
        out_specs=pl.BlockSpec((NBLK, 2 * H2), lambda i: (i, 0)),
        out_shape=jax.ShapeDtypeStruct((HALF, 2 * H2), jnp.float32),
        compiler_params=pltpu.CompilerParams(
            dimension_semantics=("arbitrary",),
        ),
    )(embT, embT, W1.T, b1.reshape(H1, 1), W2, b2.reshape(1, H2))


def _sc_gather(table, idx):
    mesh = plsc.VectorSubcoreMesh(core_axis_name="c", subcore_axis_name="s")

    @functools.partial(
        pl.kernel,
        out_type=jax.ShapeDtypeStruct((TOTAL_PAD, 2 * H2), jnp.float32),
        mesh=mesh,
        scratch_types=[
            pltpu.VMEM((CHUNK,), jnp.int32),
            pltpu.VMEM((CHUNK, H2), jnp.float32),
            pltpu.SemaphoreType.DMA,
        ],
        compiler_params=pltpu.CompilerParams(use_tc_tiling_on_sc=False),
    )
    def gather_kernel(table_hbm, idx_hbm, out_hbm, idx_v, rows_v, sem):
        wid = lax.axis_index("s") * NC + lax.axis_index("c")
        base = wid * B_PER_W

        def body(g, carry):
            off = base + g * CHUNK
            pltpu.sync_copy(idx_hbm.at[pl.ds(off, CHUNK)], idx_v)
            pltpu.async_copy(table_hbm.at[idx_v], rows_v, sem).wait()
            pltpu.sync_copy(
                rows_v, out_hbm.at[pl.ds(off, CHUNK), pl.ds(0, H2)])
            return carry

        lax.fori_loop(0, N_CHUNKS, body, 0)

    return gather_kernel(table, idx)


def kernel(batch, emb, W1, b1, W2, b2):
    r = batch.astype(jnp.int32)
    remapped = jnp.where(r < HALF, r * 2, (r - HALF) * 2 + 1)
    idx = jnp.pad(remapped, ((0, 0), (0, HIST_PAD - HIST)),
                  mode="edge").reshape(-1)
    table_out = _tc_mlp_table(emb.T, W1, b1, W2, b2)
    table_rows = table_out.reshape(-1, H2)
    out_pad = _sc_gather(table_rows, idx)
    return out_pad.reshape(BATCH, HIST_PAD, 2 * H2)[:, :HIST, :H2]
